# baseline (speedup 1.0000x reference)
import functools
import math

import jax
import jax.numpy as jnp
from jax import lax
from jax.experimental import pallas as pl
from jax.experimental.pallas import tpu as pltpu
from jax.experimental.pallas import tpu_sc as plsc

NC = 2
NS = 16
LANES = 16
EBLK = 128
PACK = 16384


def _spmm_sc(tab, pck3, vals3, zer, *, mp, hf, nblk):
    sr = mp // NS
    assert nblk % 6 == 0

    def body(tab_hbm, pck_hbm, vals_hbm, zer_hbm, z1_hbm, z2_hbm,
             pckv, valv, gbuf, sbuf, cs0, cs1, rs0, rs1, rs2,
             acc, gsem, ssem):
        c = lax.axis_index("c")
        s = lax.axis_index("s")
        base = pl.multiple_of(s * sr, 8)
        zbase = pl.multiple_of(c * mp + s * sr, 8)
        coff = c * mp
        pltpu.sync_copy(pck_hbm.at[s], pckv)
        pltpu.sync_copy(vals_hbm.at[s], valv)
        csl = [cs0, cs1]
        rsl = [rs0, rs1, rs2]

        def unpack_cols(b, dst):
            for i in range(EBLK // LANES):
                sl = pl.ds(i * LANES, LANES)
                dst[sl] = (pckv[b, sl] & (PACK - 1)) + coff

        def unpack_rows(b, dst):
            for i in range(EBLK // LANES):
                sl = pl.ds(i * LANES, LANES)
                dst[sl] = pckv[b, sl] >> 14

        def spmm_pass(tab_ref, z_out):
            pltpu.sync_copy(zer_hbm.at[pl.ds(base, sr)],
                            acc.at[pl.ds(base, sr)])
            plsc.subcore_barrier()

            def gstart(p2):
                pltpu.async_copy(tab_ref.at[csl[p2]], gbuf.at[p2], gsem)

            def gwait(p2):
                pltpu.make_async_copy(tab_ref.at[csl[p2]], gbuf.at[p2],
                                      gsem).wait()

            def sstart(p2, p3):
                pltpu.async_copy(sbuf.at[p2], acc.at[rsl[p3]], ssem,
                                 add=True)

            def swait(p2, p3):
                pltpu.make_async_copy(sbuf.at[p2], acc.at[rsl[p3]],
                                      ssem).wait()

            def scale(b, p2):
                def edge16(kk, carry2):
                    vv = valv[b, pl.ds(kk * LANES, LANES)]
                    for i in range(LANES):
                        k = kk * LANES + i
                        for j in range(hf // LANES):
                            sl = pl.ds(j * LANES, LANES)
                            sbuf[p2, k, sl] = gbuf[p2, k, sl] * vv[i]
                    return carry2

                lax.fori_loop(0, EBLK // LANES, edge16, 0, unroll=4)

            unpack_cols(0, cs0)
            unpack_cols(1, cs1)
            unpack_rows(0, rs0)
            gstart(0)
            gstart(1)

            def group(g, carry):
                for ph in range(6):
                    b = g * 6 + ph
                    p2 = ph % 2
                    p3 = ph % 3
                    gwait(p2)

                    @pl.when(b >= 2)
                    def _():
                        swait(p2, p3)

                    scale(b, p2)

                    @pl.when(b + 2 < nblk)
                    def _():
                        unpack_cols(b + 2, csl[p2])
                        gstart(p2)

                    sstart(p2, p3)

                    @pl.when(b + 1 < nblk)
                    def _():
                        unpack_rows(b + 1, rsl[(ph + 1) % 3])
                return carry

            lax.fori_loop(0, nblk // 6, group, 0)
            swait(0, (nblk - 2) % 3)
            swait(1, (nblk - 1) % 3)
            plsc.subcore_barrier()
            pltpu.sync_copy(acc.at[pl.ds(base, sr)],
                            z_out.at[pl.ds(zbase, sr)])
            plsc.subcore_barrier()

        spmm_pass(tab_hbm, z1_hbm)
        spmm_pass(z1_hbm, z2_hbm)

    f = pl.kernel(
        body,
        out_type=(jax.ShapeDtypeStruct((NC * mp, hf), jnp.float32),
                  jax.ShapeDtypeStruct((NC * mp, hf), jnp.float32)),
        mesh=plsc.VectorSubcoreMesh(core_axis_name="c", subcore_axis_name="s"),
        scratch_types=[
            pltpu.VMEM((nblk, EBLK), jnp.int32),
            pltpu.VMEM((nblk, EBLK), jnp.float32),
            pltpu.VMEM((2, EBLK, hf), jnp.float32),
            pltpu.VMEM((2, EBLK, hf), jnp.float32),
            pltpu.VMEM((EBLK,), jnp.int32),
            pltpu.VMEM((EBLK,), jnp.int32),
            pltpu.VMEM((EBLK,), jnp.int32),
            pltpu.VMEM((EBLK,), jnp.int32),
            pltpu.VMEM((EBLK,), jnp.int32),
            pltpu.VMEM_SHARED((mp, hf), jnp.float32),
            pltpu.SemaphoreType.DMA,
            pltpu.SemaphoreType.DMA,
        ],
        compiler_params=pltpu.CompilerParams(use_tc_tiling_on_sc=False),
    )
    return f(tab, pck3, vals3, zer)


def _combine_body(x0_ref, zc_ref, rc_ref, w_ref, b_ref, o_ref, *, hf):
    acc = jnp.dot(x0_ref[...], w_ref[pl.ds(0, 2 * hf), :],
                  preferred_element_type=jnp.float32)
    acc += jnp.dot(zc_ref[0], w_ref[pl.ds(2 * hf, hf), :],
                   preferred_element_type=jnp.float32)
    acc += jnp.dot(zc_ref[1], w_ref[pl.ds(3 * hf, hf), :],
                   preferred_element_type=jnp.float32)
    acc += jnp.dot(rc_ref[0], w_ref[pl.ds(4 * hf, hf), :],
                   preferred_element_type=jnp.float32)
    acc += jnp.dot(rc_ref[1], w_ref[pl.ds(5 * hf, hf), :],
                   preferred_element_type=jnp.float32)
    o_ref[...] = acc + b_ref[...]


def kernel(x, rows, cols, vals, kernel, bias):
    n, m, fin = x.shape
    filt = kernel.shape[1]
    rank = kernel.shape[0] // fin
    assert n == 1 and rank == 3 and fin % 2 == 0 and m < PACK
    hf = fin // 2

    x0 = x[0]
    mp = math.ceil(m / (8 * NS)) * 8 * NS
    tab1 = (jnp.zeros((2 * mp, hf), jnp.float32)
            .at[:m].set(x0[:, :hf]).at[mp:mp + m].set(x0[:, hf:]))

    e2 = rows.shape[0]
    eb = NS * EBLK
    nblk = 6 * math.ceil(e2 / (6 * eb))
    pad = nblk * eb - e2
    pck3 = (jnp.pad(rows, (0, pad)) * PACK
            + jnp.pad(cols, (0, pad))).reshape(NS, nblk, EBLK)
    vals3 = jnp.pad(vals, (0, pad)).reshape(NS, nblk, EBLK)
    zer = jnp.zeros((mp, hf), jnp.float32)

    z1f, r2f = _spmm_sc(tab1, pck3, vals3, zer, mp=mp, hf=hf, nblk=nblk)
    z1 = z1f.reshape(NC, mp, hf)
    r2 = r2f.reshape(NC, mp, hf)

    w = kernel.reshape(fin, rank, filt)
    w0, w1, w2 = w[:, 0, :], w[:, 1, :], w[:, 2, :]
    wbig = jnp.concatenate(
        [w0 - w2, w1[:hf], w1[hf:], 2.0 * w2[:hf], 2.0 * w2[hf:]], axis=0)
    bias2 = bias.reshape(1, filt)

    blk = 1000
    grid = m // blk
    out = pl.pallas_call(
        functools.partial(_combine_body, hf=hf),
        grid=(grid,),
        in_specs=[
            pl.BlockSpec((blk, fin), lambda i: (i, 0)),
            pl.BlockSpec((NC, blk, hf), lambda i: (0, i, 0)),
            pl.BlockSpec((NC, blk, hf), lambda i: (0, i, 0)),
            pl.BlockSpec((3 * fin, filt), lambda i: (0, 0)),
            pl.BlockSpec((1, filt), lambda i: (0, 0)),
        ],
        out_specs=pl.BlockSpec((blk, filt), lambda i: (i, 0)),
        out_shape=jax.ShapeDtypeStruct((m, filt), jnp.float32),
    )(x0, z1, r2, wbig, bias2)
    return out.reshape(1, m, filt)
